# baseline (speedup 1.0000x reference)
import functools

import jax
import jax.numpy as jnp
from jax import lax
from jax.experimental import pallas as pl
from jax.experimental.pallas import tpu as pltpu
from jax.experimental.pallas import tpu_sc as plsc

N = 10000
D = 128
G = 64
E = 320000

NC, NS = 2, 16
NW = NC * NS
C = 128
K = 80
NPHASE = 5
CPP = K // NPHASE
E_PAD = NW * K * C
ZR = 128
N_PAD = 10240
RPT = N_PAD // NS
DUMMY = N_PAD - 1

RB = 400
NB = N // RB

_HIGHEST = jax.lax.Precision.HIGHEST


def _mesh():
    return plsc.VectorSubcoreMesh(
        core_axis_name="c", subcore_axis_name="s",
        num_cores=NC, num_subcores=NS)


def _sc_agg_body(x_hbm, src_hbm, dst_hbm, zrows_hbm, zdeg_hbm,
                 agg_out, deg_out,
                 src_v, dst_v, bufA, bufB, ones_v, acc, dacc,
                 semA, semB, with_deg):
    c = lax.axis_index("c")
    s = lax.axis_index("s")
    wid = c * NS + s

    base = s * RPT
    for k in range(RPT // ZR):
        pltpu.sync_copy(zrows_hbm, acc.at[pl.ds(base + k * ZR, ZR)])
    if with_deg:
        pltpu.sync_copy(zdeg_hbm, dacc.at[pl.ds(base, RPT)])
        for k in range(C // 16):
            ones_v[pl.ds(k * 16, 16)] = jnp.ones((16,), jnp.float32)
    plsc.subcore_barrier()

    bufs = (bufA, bufB)
    sems = (semA, semB)

    for p in range(NPHASE):
        pltpu.sync_copy(src_hbm.at[wid, pl.ds(p * CPP, CPP)], src_v)
        pltpu.sync_copy(dst_hbm.at[wid, pl.ds(p * CPP, CPP)], dst_v)

        pltpu.async_copy(x_hbm.at[src_v.at[0]], bufs[0], sems[0])

        def step(i, _):
            for b in range(2):
                j = 2 * i + b
                nxt = j + 1

                @pl.when(nxt < CPP)
                def _():
                    pltpu.async_copy(
                        x_hbm.at[src_v.at[nxt]], bufs[(b + 1) % 2],
                        sems[(b + 1) % 2])

                pltpu.make_async_copy(
                    x_hbm.at[src_v.at[j]], bufs[b], sems[b]).wait()
                pltpu.sync_copy(bufs[b], acc.at[dst_v.at[j]], add=True)
                if with_deg:
                    pltpu.sync_copy(ones_v, dacc.at[dst_v.at[j]], add=True)
            return 0

        lax.fori_loop(0, CPP // 2, step, 0)
    plsc.subcore_barrier()

    pltpu.sync_copy(acc.at[pl.ds(base, RPT)],
                    agg_out.at[c, pl.ds(base, RPT)])
    if with_deg:
        pltpu.sync_copy(dacc.at[pl.ds(base, RPT)],
                        deg_out.at[c, pl.ds(base, RPT)])


def _make_sc_agg(with_deg):
    out_type = [jax.ShapeDtypeStruct((NC, N_PAD, D), jnp.float32)]
    if with_deg:
        out_type.append(jax.ShapeDtypeStruct((NC, N_PAD), jnp.float32))
    scratch = [
        pltpu.VMEM((CPP, C), jnp.int32),
        pltpu.VMEM((CPP, C), jnp.int32),
        pltpu.VMEM((C, D), jnp.float32),
        pltpu.VMEM((C, D), jnp.float32),
        pltpu.VMEM((C,), jnp.float32),
        pltpu.VMEM_SHARED((N_PAD, D), jnp.float32),
        pltpu.VMEM_SHARED((N_PAD,), jnp.float32),
        pltpu.SemaphoreType.DMA,
        pltpu.SemaphoreType.DMA,
    ]

    @functools.partial(pl.kernel, out_type=tuple(out_type), mesh=_mesh(),
                       scratch_types=scratch)
    def k(x_hbm, src_hbm, dst_hbm, zrows_hbm, zdeg_hbm, *rest):
        if with_deg:
            agg_out, deg_out = rest[0], rest[1]
            scr = rest[2:]
        else:
            agg_out, deg_out = rest[0], None
            scr = rest[1:]
        (src_v, dst_v, bufA, bufB, ones_v, acc, dacc, semA, semB) = scr
        _sc_agg_body(x_hbm, src_hbm, dst_hbm, zrows_hbm, zdeg_hbm,
                     agg_out, deg_out,
                     src_v, dst_v, bufA, bufB, ones_v, acc, dacc,
                     semA, semB, with_deg)

    return k


@functools.lru_cache(maxsize=None)
def _get_sc_agg(with_deg):
    return _make_sc_agg(with_deg)


def _dense_body(agg_ref, deg_ref, x_ref, wl_ref, bl_ref, wr_ref, h_ref):
    a = agg_ref[0] + agg_ref[1]
    dg = deg_ref[0] + deg_ref[1]
    agg = a / jnp.maximum(dg, 1.0)
    out = (lax.dot_general(agg, wl_ref[...], (((1,), (1,)), ((), ())),
                           precision=_HIGHEST)
           + lax.dot_general(x_ref[...], wr_ref[...], (((1,), (1,)), ((), ())),
                             precision=_HIGHEST)
           + bl_ref[...][None, :])
    nrm = jnp.sqrt(jnp.sum(out * out, axis=1, keepdims=True))
    h_ref[...] = jnp.maximum(out / jnp.maximum(nrm, 1e-12), 0.0)


def _dense1(aggp, degp, x, Wl, bl, Wr):
    return pl.pallas_call(
        _dense_body,
        grid=(NB,),
        in_specs=[
            pl.BlockSpec((NC, RB, D), lambda i: (0, i, 0)),
            pl.BlockSpec((NC, RB, 1), lambda i: (0, i, 0)),
            pl.BlockSpec((RB, D), lambda i: (i, 0)),
            pl.BlockSpec((D, D), lambda i: (0, 0)),
            pl.BlockSpec((D,), lambda i: (0,)),
            pl.BlockSpec((D, D), lambda i: (0, 0)),
        ],
        out_specs=pl.BlockSpec((RB, D), lambda i: (i, 0)),
        out_shape=jax.ShapeDtypeStruct((N, D), jnp.float32),
    )(aggp, degp, x, Wl, bl, Wr)


def _dense2_body(agg_ref, deg_ref, h_ref, wl_ref, bl_ref, wr_ref,
                 wlin_ref, blin_ref, batch_ref, out_ref, pool_ref):
    i = pl.program_id(0)

    @pl.when(i == 0)
    def _():
        pool_ref[...] = jnp.full((G, D), -jnp.inf, jnp.float32)

    a = agg_ref[0] + agg_ref[1]
    dg = deg_ref[0] + deg_ref[1]
    agg = a / jnp.maximum(dg, 1.0)
    out = (lax.dot_general(agg, wl_ref[...], (((1,), (1,)), ((), ())),
                           precision=_HIGHEST)
           + lax.dot_general(h_ref[...], wr_ref[...], (((1,), (1,)), ((), ())),
                             precision=_HIGHEST)
           + bl_ref[...][None, :])
    nrm = jnp.sqrt(jnp.sum(out * out, axis=1, keepdims=True))
    h2 = jnp.maximum(out / jnp.maximum(nrm, 1e-12), 0.0)

    batch_blk = batch_ref[...]
    g_lo = jnp.min(batch_blk)
    g_hi = jnp.max(batch_blk)

    def body(g, carry):
        m = batch_blk == g
        v = jnp.max(jnp.where(m, h2, -jnp.inf), axis=0)
        cur = pool_ref[pl.ds(g, 1), :]
        pool_ref[pl.ds(g, 1), :] = jnp.maximum(cur, v[None, :])
        return carry

    lax.fori_loop(g_lo, g_hi + 1, body, 0)

    @pl.when(i == NB - 1)
    def _():
        out_ref[...] = (lax.dot_general(
            pool_ref[...], wlin_ref[...], (((1,), (1,)), ((), ())),
            precision=_HIGHEST) + blin_ref[...][None, :])


def _dense2(aggp, degp, h1, Wl, bl, Wr, Wlin, blin, batch3):
    nout = Wlin.shape[0]
    return pl.pallas_call(
        _dense2_body,
        grid=(NB,),
        in_specs=[
            pl.BlockSpec((NC, RB, D), lambda i: (0, i, 0)),
            pl.BlockSpec((NC, RB, 1), lambda i: (0, i, 0)),
            pl.BlockSpec((RB, D), lambda i: (i, 0)),
            pl.BlockSpec((D, D), lambda i: (0, 0)),
            pl.BlockSpec((D,), lambda i: (0,)),
            pl.BlockSpec((D, D), lambda i: (0, 0)),
            pl.BlockSpec((nout, D), lambda i: (0, 0)),
            pl.BlockSpec((nout,), lambda i: (0,)),
            pl.BlockSpec((RB, 1), lambda i: (i, 0)),
        ],
        out_specs=pl.BlockSpec((G, nout), lambda i: (0, 0)),
        out_shape=jax.ShapeDtypeStruct((G, nout), jnp.float32),
        scratch_shapes=[pltpu.VMEM((G, D), jnp.float32)],
    )(aggp, degp, h1, Wl, bl, Wr, Wlin, blin, batch3)


def kernel(x, edge_index, batch, Wl1, bl1, Wr1, Wl2, bl2, Wr2, Wlin, blin):
    src = edge_index[0].astype(jnp.int32)
    dst = edge_index[1].astype(jnp.int32)
    pad = E_PAD - E
    src3 = jnp.concatenate([src, jnp.zeros((pad,), jnp.int32)]).reshape(NW, K, C)
    dummy = N + (jnp.arange(pad, dtype=jnp.int32) % (N_PAD - N))
    dst3 = jnp.concatenate([dst, dummy]).reshape(NW, K, C)
    zrows = jnp.zeros((ZR, D), jnp.float32)
    zdeg = jnp.zeros((RPT,), jnp.float32)

    aggp1, degp = _get_sc_agg(True)(x, src3, dst3, zrows, zdeg)
    degp = degp.reshape(NC, N_PAD, 1)
    h1 = _dense1(aggp1, degp, x, Wl1, bl1, Wr1)
    (aggp2,) = _get_sc_agg(False)(h1, src3, dst3, zrows, zdeg)
    batch3 = batch.astype(jnp.int32).reshape(N, 1)
    return _dense2(aggp2, degp, h1, Wl2, bl2, Wr2, Wlin, blin, batch3)


# speedup vs baseline: 1.0554x; 1.0554x over previous
import functools

import jax
import jax.numpy as jnp
from jax import lax
from jax.experimental import pallas as pl
from jax.experimental.pallas import tpu as pltpu
from jax.experimental.pallas import tpu_sc as plsc

N = 10000
D = 128
G = 64
E = 320000

NC, NS = 2, 16
NW = NC * NS
C = 128
CPP = 16
K0 = 128
K1 = 32
NPH0 = K0 // CPP
NCHUNKS = NS * (K0 + K1)
E_PAD = NCHUNKS * C
N_PAD = 10240
RPT = N_PAD // NS
DUMMY = N_PAD - 1

RB = 400
NB = N // RB

_HIGHEST = jax.lax.Precision.HIGHEST


def _mesh():
    return plsc.VectorSubcoreMesh(
        core_axis_name="c", subcore_axis_name="s",
        num_cores=NC, num_subcores=NS)


def _sc_agg_body(x_hbm, src_hbm, dst_hbm, zdeg_hbm,
                 agg_out, deg_out,
                 src_v, dst_v, bufA, bufB, ones_v, acc, dacc,
                 semA, semB, with_deg):
    c = lax.axis_index("c")
    s = lax.axis_index("s")
    my_k = jnp.where(c == 0, K0, K1)
    off = jnp.where(c == 0, s * K0, NS * K0 + s * K1)

    for r in range(C):
        for q in range(D // 16):
            bufA[r, pl.ds(q * 16, 16)] = jnp.zeros((16,), jnp.float32)
    base = s * RPT
    for k in range(RPT // C):
        pltpu.sync_copy(bufA, acc.at[pl.ds(base + k * C, C)])
    if with_deg:
        pltpu.sync_copy(zdeg_hbm, dacc.at[pl.ds(base, RPT)])
        for k in range(C // 16):
            ones_v[pl.ds(k * 16, 16)] = jnp.ones((16,), jnp.float32)
    plsc.subcore_barrier()

    bufs = (bufA, bufB)
    sems = (semA, semB)

    for p in range(NPH0):
        @pl.when(p * CPP < my_k)
        def _phase():
            pltpu.sync_copy(src_hbm.at[pl.ds(off + p * CPP, CPP)], src_v)
            pltpu.sync_copy(dst_hbm.at[pl.ds(off + p * CPP, CPP)], dst_v)

            pltpu.async_copy(x_hbm.at[src_v.at[0]], bufs[0], sems[0])

            def step(i, _):
                for b in range(2):
                    j = 2 * i + b
                    nxt = j + 1

                    @pl.when(nxt < CPP)
                    def _():
                        pltpu.async_copy(
                            x_hbm.at[src_v.at[nxt]], bufs[(b + 1) % 2],
                            sems[(b + 1) % 2])

                    pltpu.make_async_copy(
                        x_hbm.at[src_v.at[j]], bufs[b], sems[b]).wait()
                    pltpu.sync_copy(bufs[b], acc.at[dst_v.at[j]], add=True)
                    if with_deg:
                        pltpu.sync_copy(ones_v, dacc.at[dst_v.at[j]],
                                        add=True)
                return 0

            lax.fori_loop(0, CPP // 2, step, 0)
    plsc.subcore_barrier()

    pltpu.sync_copy(acc.at[pl.ds(base, RPT)],
                    agg_out.at[c, pl.ds(base, RPT)])
    if with_deg:
        pltpu.sync_copy(dacc.at[pl.ds(base, RPT)],
                        deg_out.at[c, pl.ds(base, RPT)])


def _make_sc_agg(with_deg):
    out_type = [jax.ShapeDtypeStruct((NC, N_PAD, D), jnp.float32)]
    if with_deg:
        out_type.append(jax.ShapeDtypeStruct((NC, N_PAD), jnp.float32))
    scratch = [
        pltpu.VMEM((CPP, C), jnp.int32),
        pltpu.VMEM((CPP, C), jnp.int32),
        pltpu.VMEM((C, D), jnp.float32),
        pltpu.VMEM((C, D), jnp.float32),
        pltpu.VMEM((C,), jnp.float32),
        pltpu.VMEM_SHARED((N_PAD, D), jnp.float32),
        pltpu.VMEM_SHARED((N_PAD,), jnp.float32),
        pltpu.SemaphoreType.DMA,
        pltpu.SemaphoreType.DMA,
    ]

    @functools.partial(pl.kernel, out_type=tuple(out_type), mesh=_mesh(),
                       scratch_types=scratch)
    def k(x_hbm, src_hbm, dst_hbm, zdeg_hbm, *rest):
        if with_deg:
            agg_out, deg_out = rest[0], rest[1]
            scr = rest[2:]
        else:
            agg_out, deg_out = rest[0], None
            scr = rest[1:]
        (src_v, dst_v, bufA, bufB, ones_v, acc, dacc, semA, semB) = scr
        _sc_agg_body(x_hbm, src_hbm, dst_hbm, zdeg_hbm,
                     agg_out, deg_out,
                     src_v, dst_v, bufA, bufB, ones_v, acc, dacc,
                     semA, semB, with_deg)

    return k


@functools.lru_cache(maxsize=None)
def _get_sc_agg(with_deg):
    return _make_sc_agg(with_deg)


def _dense_body(agg_ref, deg_ref, x_ref, wl_ref, bl_ref, wr_ref, h_ref):
    a = agg_ref[0] + agg_ref[1]
    dg = deg_ref[0] + deg_ref[1]
    agg = a / jnp.maximum(dg, 1.0)
    out = (lax.dot_general(agg, wl_ref[...], (((1,), (1,)), ((), ())),
                           precision=_HIGHEST)
           + lax.dot_general(x_ref[...], wr_ref[...], (((1,), (1,)), ((), ())),
                             precision=_HIGHEST)
           + bl_ref[...][None, :])
    nrm = jnp.sqrt(jnp.sum(out * out, axis=1, keepdims=True))
    h_ref[...] = jnp.maximum(out / jnp.maximum(nrm, 1e-12), 0.0)


def _dense1(aggp, degp, x, Wl, bl, Wr):
    return pl.pallas_call(
        _dense_body,
        grid=(NB,),
        in_specs=[
            pl.BlockSpec((NC, RB, D), lambda i: (0, i, 0)),
            pl.BlockSpec((NC, RB, 1), lambda i: (0, i, 0)),
            pl.BlockSpec((RB, D), lambda i: (i, 0)),
            pl.BlockSpec((D, D), lambda i: (0, 0)),
            pl.BlockSpec((D,), lambda i: (0,)),
            pl.BlockSpec((D, D), lambda i: (0, 0)),
        ],
        out_specs=pl.BlockSpec((RB, D), lambda i: (i, 0)),
        out_shape=jax.ShapeDtypeStruct((N, D), jnp.float32),
    )(aggp, degp, x, Wl, bl, Wr)


def _dense2_body(agg_ref, deg_ref, h_ref, wl_ref, bl_ref, wr_ref,
                 wlin_ref, blin_ref, batch_ref, out_ref, pool_ref):
    i = pl.program_id(0)

    @pl.when(i == 0)
    def _():
        pool_ref[...] = jnp.full((G, D), -jnp.inf, jnp.float32)

    a = agg_ref[0] + agg_ref[1]
    dg = deg_ref[0] + deg_ref[1]
    agg = a / jnp.maximum(dg, 1.0)
    out = (lax.dot_general(agg, wl_ref[...], (((1,), (1,)), ((), ())),
                           precision=_HIGHEST)
           + lax.dot_general(h_ref[...], wr_ref[...], (((1,), (1,)), ((), ())),
                             precision=_HIGHEST)
           + bl_ref[...][None, :])
    nrm = jnp.sqrt(jnp.sum(out * out, axis=1, keepdims=True))
    h2 = jnp.maximum(out / jnp.maximum(nrm, 1e-12), 0.0)

    batch_blk = batch_ref[...]
    g_lo = jnp.min(batch_blk)
    g_hi = jnp.max(batch_blk)

    def body(g, carry):
        m = batch_blk == g
        v = jnp.max(jnp.where(m, h2, -jnp.inf), axis=0)
        cur = pool_ref[pl.ds(g, 1), :]
        pool_ref[pl.ds(g, 1), :] = jnp.maximum(cur, v[None, :])
        return carry

    lax.fori_loop(g_lo, g_hi + 1, body, 0)

    @pl.when(i == NB - 1)
    def _():
        out_ref[...] = (lax.dot_general(
            pool_ref[...], wlin_ref[...], (((1,), (1,)), ((), ())),
            precision=_HIGHEST) + blin_ref[...][None, :])


def _dense2(aggp, degp, h1, Wl, bl, Wr, Wlin, blin, batch3):
    nout = Wlin.shape[0]
    return pl.pallas_call(
        _dense2_body,
        grid=(NB,),
        in_specs=[
            pl.BlockSpec((NC, RB, D), lambda i: (0, i, 0)),
            pl.BlockSpec((NC, RB, 1), lambda i: (0, i, 0)),
            pl.BlockSpec((RB, D), lambda i: (i, 0)),
            pl.BlockSpec((D, D), lambda i: (0, 0)),
            pl.BlockSpec((D,), lambda i: (0,)),
            pl.BlockSpec((D, D), lambda i: (0, 0)),
            pl.BlockSpec((nout, D), lambda i: (0, 0)),
            pl.BlockSpec((nout,), lambda i: (0,)),
            pl.BlockSpec((RB, 1), lambda i: (i, 0)),
        ],
        out_specs=pl.BlockSpec((G, nout), lambda i: (0, 0)),
        out_shape=jax.ShapeDtypeStruct((G, nout), jnp.float32),
        scratch_shapes=[pltpu.VMEM((G, D), jnp.float32)],
    )(aggp, degp, h1, Wl, bl, Wr, Wlin, blin, batch3)


def kernel(x, edge_index, batch, Wl1, bl1, Wr1, Wl2, bl2, Wr2, Wlin, blin):
    src = edge_index[0].astype(jnp.int32)
    dst = edge_index[1].astype(jnp.int32)
    pad = E_PAD - E
    src2 = jnp.concatenate(
        [src, jnp.zeros((pad,), jnp.int32)]).reshape(NCHUNKS, C)
    dummy = N + (jnp.arange(pad, dtype=jnp.int32) % (N_PAD - N))
    dst2 = jnp.concatenate([dst, dummy]).reshape(NCHUNKS, C)
    zdeg = jnp.zeros((RPT,), jnp.float32)

    aggp1, degp = _get_sc_agg(True)(x, src2, dst2, zdeg)
    degp = degp.reshape(NC, N_PAD, 1)
    h1 = _dense1(aggp1, degp, x, Wl1, bl1, Wr1)
    (aggp2,) = _get_sc_agg(False)(h1, src2, dst2, zdeg)
    batch3 = batch.astype(jnp.int32).reshape(N, 1)
    return _dense2(aggp2, degp, h1, Wl2, bl2, Wr2, Wlin, blin, batch3)
